# Initial kernel scaffold; baseline (speedup 1.0000x reference)
import jax
import jax.numpy as jnp
from jax.experimental import pallas as pl


def kernel(x, edge_indices, edge_types, W1, root1, b1, W2, root2, b2):
    raise NotImplementedError("write your pallas kernel here")


# speedup vs baseline: 10.5681x; 10.5681x over previous
import functools

import jax
import jax.numpy as jnp
from jax import lax
from jax.experimental import pallas as pl
from jax.experimental.pallas import tpu as pltpu
from jax.experimental.pallas import tpu_sc as plsc

LANES = 16
NSUB = 16
NCORE = 2
C = 128


def _xw_body(x_ref, w_ref, o_ref):
    o_ref[...] = jnp.dot(x_ref[...], w_ref[0], preferred_element_type=jnp.float32)


def _compute_xw(x, W):
    R, D, _ = W.shape
    N = x.shape[0]
    NB = 1000
    nb = N // NB
    return pl.pallas_call(
        _xw_body,
        grid=(R, nb),
        in_specs=[
            pl.BlockSpec((NB, D), lambda r, i: (i, 0)),
            pl.BlockSpec((1, D, D), lambda r, i: (r, 0, 0)),
        ],
        out_specs=pl.BlockSpec((NB, D), lambda r, i: (r * nb + i, 0)),
        out_shape=jax.ShapeDtypeStruct((R * N, D), jnp.float32),
    )(x, W)


def _root_body(x_ref, w_ref, b_ref, o_ref):
    o_ref[...] = (
        jnp.dot(x_ref[...], w_ref[...], preferred_element_type=jnp.float32)
        + b_ref[...]
    )


def _compute_root(x, root, b):
    N, D = x.shape
    NB = 1000
    return pl.pallas_call(
        _root_body,
        grid=(N // NB,),
        in_specs=[
            pl.BlockSpec((NB, D), lambda i: (i, 0)),
            pl.BlockSpec((D, D), lambda i: (0, 0)),
            pl.BlockSpec((D,), lambda i: (0,)),
        ],
        out_specs=pl.BlockSpec((NB, D), lambda i: (i, 0)),
        out_shape=jax.ShapeDtypeStruct((N, D), jnp.float32),
    )(x, root, b)


def _combine_body(p_ref, rx_ref, o_ref):
    o_ref[...] = jnp.maximum(p_ref[0] + p_ref[1] + rx_ref[...], 0.0)


def _combine(partials, rootx):
    _, N, D = partials.shape
    NB = 1000
    return pl.pallas_call(
        _combine_body,
        grid=(N // NB,),
        in_specs=[
            pl.BlockSpec((2, NB, D), lambda i: (0, i, 0)),
            pl.BlockSpec((NB, D), lambda i: (i, 0)),
        ],
        out_specs=pl.BlockSpec((NB, D), lambda i: (i, 0)),
        out_shape=jax.ShapeDtypeStruct((N, D), jnp.float32),
    )(partials, rootx)


def _sc_pass(xw, src, dst, typ, N, R, D, E):
    NCH = E // C
    rows_t = (N // NSUB) // 8 * 8
    rows_tail = N - rows_t * NSUB
    cnt_t = -(-(N * R) // (NSUB * LANES)) * LANES
    NRP = cnt_t * NSUB

    mesh = plsc.VectorSubcoreMesh(core_axis_name="c", subcore_axis_name="s")

    @functools.partial(
        pl.kernel,
        mesh=mesh,
        out_type=jax.ShapeDtypeStruct((2, N, D), jnp.float32),
        scratch_types=[
            pltpu.VMEM_SHARED((NRP,), jnp.float32),
            pltpu.VMEM_SHARED((N, D), jnp.float32),
            pltpu.VMEM((C,), jnp.int32),
            pltpu.VMEM((C,), jnp.int32),
            pltpu.VMEM((C,), jnp.int32),
            pltpu.VMEM((C,), jnp.int32),
            pltpu.VMEM((C,), jnp.int32),
            pltpu.VMEM((C,), jnp.float32),
            pltpu.VMEM((C, D), jnp.float32),
            pltpu.VMEM((cnt_t,), jnp.float32),
            pltpu.SemaphoreType.DMA,
            pltpu.SemaphoreType.DMA,
        ],
    )
    def sc_fn(xw_h, src_h, dst_h, typ_h, out_h,
              cnt_sh, acc_sh, s_v, d_v, t_v, g_v, w_i, w_v, rows_v, cb_v,
              sem_a, sem_b):
        core = lax.axis_index("c")
        sub = lax.axis_index("s")
        wid = core * NSUB + sub

        def _zrow(i, _):
            for j in range(D // LANES):
                rows_v[i, pl.ds(j * LANES, LANES)] = jnp.zeros((LANES,), jnp.float32)
            return 0
        lax.fori_loop(0, C, _zrow, 0)

        def _zcnt(i, _):
            cb_v[pl.ds(i * LANES, LANES)] = jnp.zeros((LANES,), jnp.float32)
            return 0
        lax.fori_loop(0, cnt_t // LANES, _zcnt, 0)

        for j in range(C // LANES):
            w_v[pl.ds(j * LANES, LANES)] = jnp.ones((LANES,), jnp.float32)

        pltpu.sync_copy(cb_v, cnt_sh.at[pl.ds(sub * cnt_t, cnt_t)])
        r0 = sub * rows_t
        off = 0
        while off < rows_t:
            nblk = min(C, rows_t - off)
            pltpu.sync_copy(rows_v.at[pl.ds(0, nblk)],
                            acc_sh.at[pl.ds(r0 + off, nblk)])
            off += nblk
        if rows_tail:
            @pl.when(sub == NSUB - 1)
            def _zero_tail():
                pltpu.sync_copy(rows_v.at[pl.ds(0, rows_tail)],
                                acc_sh.at[pl.ds(rows_t * NSUB, rows_tail)])
        plsc.subcore_barrier()

        ntrips_a = (NCH - sub + NSUB - 1) // NSUB

        def _count(k, _):
            base = pl.multiple_of((sub + k * NSUB) * C, C)
            pltpu.sync_copy(dst_h.at[pl.ds(base, C)], d_v)
            pltpu.sync_copy(typ_h.at[pl.ds(base, C)], t_v)
            for g in range(C // LANES):
                sl = pl.ds(g * LANES, LANES)
                w_i[sl] = d_v[sl] * R + t_v[sl]
            pltpu.sync_copy(w_v, cnt_sh.at[w_i], add=True)
            return 0
        lax.fori_loop(0, ntrips_a, _count, 0)
        plsc.subcore_barrier()

        pltpu.sync_copy(cnt_sh.at[pl.ds(sub * cnt_t, cnt_t)], cb_v)

        def _inv(i, _):
            sl = pl.ds(i * LANES, LANES)
            cb_v[sl] = 1.0 / jnp.maximum(cb_v[sl], 1.0)
            return 0
        lax.fori_loop(0, cnt_t // LANES, _inv, 0)
        pltpu.sync_copy(cb_v, cnt_sh.at[pl.ds(sub * cnt_t, cnt_t)])
        plsc.subcore_barrier()

        NW = NCORE * NSUB
        ntrips_b = (NCH - wid + NW - 1) // NW

        def _edges(k, _):
            base = pl.multiple_of((wid + k * NW) * C, C)
            pltpu.sync_copy(src_h.at[pl.ds(base, C)], s_v)
            pltpu.sync_copy(dst_h.at[pl.ds(base, C)], d_v)
            pltpu.sync_copy(typ_h.at[pl.ds(base, C)], t_v)
            for g in range(C // LANES):
                sl = pl.ds(g * LANES, LANES)
                g_v[sl] = t_v[sl] * N + s_v[sl]
                w_i[sl] = d_v[sl] * R + t_v[sl]
            cp_rows = pltpu.async_copy(xw_h.at[g_v], rows_v, sem_a)
            cp_wts = pltpu.async_copy(cnt_sh.at[w_i], w_v, sem_b)
            cp_wts.wait()
            cp_rows.wait()

            def _scale(g, _):
                w16 = w_v[pl.ds(g * LANES, LANES)]
                for lane in range(LANES):
                    e = g * LANES + lane
                    wgt = w16[lane]
                    for j in range(D // LANES):
                        sl = pl.ds(j * LANES, LANES)
                        rows_v[e, sl] = rows_v[e, sl] * wgt
                return 0
            lax.fori_loop(0, C // LANES, _scale, 0)
            pltpu.sync_copy(rows_v, acc_sh.at[d_v], add=True)
            return 0
        lax.fori_loop(0, ntrips_b, _edges, 0)
        plsc.subcore_barrier()

        pltpu.sync_copy(acc_sh.at[pl.ds(r0, rows_t)],
                        out_h.at[core, pl.ds(r0, rows_t)])
        if rows_tail:
            @pl.when(sub == NSUB - 1)
            def _write_tail():
                pltpu.sync_copy(acc_sh.at[pl.ds(rows_t * NSUB, rows_tail)],
                                out_h.at[core, pl.ds(rows_t * NSUB, rows_tail)])

    return sc_fn(xw, src, dst, typ)


def kernel(x, edge_indices, edge_types, W1, root1, b1, W2, root2, b2):
    N, D = x.shape
    R = W1.shape[0]
    E = edge_indices.shape[2]

    def layer(h, src, dst, typ, W, root, b):
        xw = _compute_xw(h, W)
        rootx = _compute_root(h, root, b)
        partials = _sc_pass(xw, src, dst, typ, N=N, R=R, D=D, E=E)
        return _combine(partials, rootx)

    h = layer(x, edge_indices[0, 0], edge_indices[0, 1], edge_types[0],
              W1, root1, b1)
    out = layer(h, edge_indices[1, 0], edge_indices[1, 1], edge_types[1],
                W2, root2, b2)
    return out
